# Initial kernel scaffold; baseline (speedup 1.0000x reference)
import jax
import jax.numpy as jnp
from jax.experimental import pallas as pl


def kernel(pos, edge_index, edge_shift, lattice, batch):
    raise NotImplementedError("write your pallas kernel here")


# speedup vs baseline: 77.9482x; 77.9482x over previous
import functools

import jax
import jax.numpy as jnp
from jax import lax
from jax.experimental import pallas as pl
from jax.experimental.pallas import tpu as pltpu
from jax.experimental.pallas import tpu_sc as plsc

_NW = 32
_L = 16


def _rsqrt(x):
    bits = plsc.bitcast(x, jnp.int32)
    y = plsc.bitcast(jnp.full((_L,), 0x5F3759DF, jnp.int32) - (bits >> 1),
                     jnp.float32)
    half = x * 0.5
    for _ in range(3):
        y = y * (1.5 - half * y * y)
    return y


@functools.lru_cache(maxsize=None)
def _make_sc_kernel(N, E, G, C):
    EW = E // _NW
    R = C // _L
    NCH = EW // C
    mesh = plsc.VectorSubcoreMesh(core_axis_name="c", subcore_axis_name="s")

    f32 = jnp.float32
    i32 = jnp.int32

    @functools.partial(
        pl.kernel,
        mesh=mesh,
        out_type=jax.ShapeDtypeStruct((E,), f32),
        compiler_params=pltpu.CompilerParams(needs_layout_passes=False),
        scratch_types=[
            pltpu.VMEM((G * 9,), f32),
            pltpu.VMEM((C,), i32),
            pltpu.VMEM((C,), i32),
            pltpu.VMEM((C,), i32),
            pltpu.VMEM((C,), f32),
            pltpu.VMEM((C,), f32),
            pltpu.VMEM((C,), f32),
            pltpu.VMEM((C,), f32),
            pltpu.VMEM((C,), f32),
            pltpu.VMEM((C,), f32),
            pltpu.VMEM((C,), f32),
            pltpu.VMEM((C,), f32),
            pltpu.VMEM((C,), f32),
            pltpu.VMEM((C,), f32),
            pltpu.SemaphoreType.DMA,
            pltpu.SemaphoreType.DMA,
            pltpu.SemaphoreType.DMA,
        ],
    )
    def k(px_hbm, py_hbm, pz_hbm, bat_hbm, lat_hbm,
          src_hbm, dst_hbm, s0_hbm, s1_hbm, s2_hbm, out_hbm,
          lat_v, src_v, dst_v, b_v,
          sx_v, sy_v, sz_v, dx_v, dy_v, dz_v,
          h0_v, h1_v, h2_v, out_v,
          sem_lin, sem_g, sem_out):
        wid = lax.axis_index("s") * 2 + lax.axis_index("c")
        pltpu.sync_copy(lat_hbm, lat_v)

        def chunk_body(i, carry):
            e0 = wid * EW + i * C
            sl = pl.ds(e0, C)
            lin = [
                pltpu.async_copy(src_hbm.at[sl], src_v, sem_lin),
                pltpu.async_copy(dst_hbm.at[sl], dst_v, sem_lin),
                pltpu.async_copy(s0_hbm.at[sl], h0_v, sem_lin),
                pltpu.async_copy(s1_hbm.at[sl], h1_v, sem_lin),
                pltpu.async_copy(s2_hbm.at[sl], h2_v, sem_lin),
            ]
            lin[0].wait()
            lin[1].wait()
            gth = [
                pltpu.async_copy(px_hbm.at[src_v], sx_v, sem_g),
                pltpu.async_copy(py_hbm.at[src_v], sy_v, sem_g),
                pltpu.async_copy(pz_hbm.at[src_v], sz_v, sem_g),
                pltpu.async_copy(bat_hbm.at[src_v], b_v, sem_g),
                pltpu.async_copy(px_hbm.at[dst_v], dx_v, sem_g),
                pltpu.async_copy(py_hbm.at[dst_v], dy_v, sem_g),
                pltpu.async_copy(pz_hbm.at[dst_v], dz_v, sem_g),
            ]
            for g_ in gth:
                g_.wait()
            lin[2].wait()
            lin[3].wait()
            lin[4].wait()

            def grp(g, carry2):
                q = pl.ds(g * _L, _L)
                b9 = b_v[q] * 9
                lat = [plsc.load_gather(lat_v, [b9 + kk]) for kk in range(9)]
                sh0 = h0_v[q]
                sh1 = h1_v[q]
                sh2 = h2_v[q]
                vx = dx_v[q] - sx_v[q] + sh0 * lat[0] + sh1 * lat[3] \
                    + sh2 * lat[6]
                vy = dy_v[q] - sy_v[q] + sh0 * lat[1] + sh1 * lat[4] \
                    + sh2 * lat[7]
                vz = dz_v[q] - sz_v[q] + sh0 * lat[2] + sh1 * lat[5] \
                    + sh2 * lat[8]
                n2 = jnp.maximum(vx * vx + vy * vy + vz * vz, 1e-30)
                out_v[q] = n2 * _rsqrt(n2)
                return carry2

            lax.fori_loop(0, R, grp, 0)
            pltpu.async_copy(out_v, out_hbm.at[sl], sem_out).wait()
            return carry

        lax.fori_loop(0, NCH, chunk_body, 0)

    return k


def kernel(pos, edge_index, edge_shift, lattice, batch):
    N = pos.shape[0]
    E = edge_index.shape[1]
    G = lattice.shape[0]
    latf = lattice.reshape(G * 9)
    ew = E // _NW
    c = min(4000, ew)
    c -= c % _L
    while c > _L and ew % c:
        c -= _L
    k = _make_sc_kernel(N, E, G, c)
    return k(pos[:, 0], pos[:, 1], pos[:, 2], batch, latf,
             edge_index[0], edge_index[1],
             edge_shift[:, 0], edge_shift[:, 1], edge_shift[:, 2])


# speedup vs baseline: 80.8704x; 1.0375x over previous
import functools

import jax
import jax.numpy as jnp
from jax import lax
from jax.experimental import pallas as pl
from jax.experimental.pallas import tpu as pltpu
from jax.experimental.pallas import tpu_sc as plsc

_NW = 32
_L = 16


def _rsqrt(x):
    bits = plsc.bitcast(x, jnp.int32)
    y = plsc.bitcast(jnp.full((_L,), 0x5F3759DF, jnp.int32) - (bits >> 1),
                     jnp.float32)
    half = x * 0.5
    for _ in range(3):
        y = y * (1.5 - half * y * y)
    return y


@functools.lru_cache(maxsize=None)
def _make_sc_kernel(N, E, G, C):
    EW = E // _NW
    R = C // _L
    NCH = EW // C
    mesh = plsc.VectorSubcoreMesh(core_axis_name="c", subcore_axis_name="s")

    f32 = jnp.float32
    i32 = jnp.int32

    @functools.partial(
        pl.kernel,
        mesh=mesh,
        out_type=jax.ShapeDtypeStruct((E,), f32),
        compiler_params=pltpu.CompilerParams(needs_layout_passes=False),
        scratch_types=[
            pltpu.VMEM((G * 9,), f32),
            pltpu.VMEM((N,), i32),
            pltpu.VMEM((C,), i32),
            pltpu.VMEM((C,), i32),
            pltpu.VMEM((C,), f32),
            pltpu.VMEM((C,), f32),
            pltpu.VMEM((C,), f32),
            pltpu.VMEM((C,), f32),
            pltpu.VMEM((C,), f32),
            pltpu.VMEM((C,), f32),
            pltpu.VMEM((C,), f32),
            pltpu.VMEM((C,), f32),
            pltpu.VMEM((C,), f32),
            pltpu.VMEM((C,), f32),
            pltpu.SemaphoreType.DMA,
            pltpu.SemaphoreType.DMA,
            pltpu.SemaphoreType.DMA,
        ],
    )
    def k(px_hbm, py_hbm, pz_hbm, bat_hbm, lat_hbm,
          src_hbm, dst_hbm, s0_hbm, s1_hbm, s2_hbm, out_hbm,
          lat_v, bat_v, src_v, dst_v,
          sx_v, sy_v, sz_v, dx_v, dy_v, dz_v,
          h0_v, h1_v, h2_v, out_v,
          sem_lin, sem_g, sem_out):
        wid = lax.axis_index("s") * 2 + lax.axis_index("c")
        pltpu.sync_copy(lat_hbm, lat_v)
        pltpu.sync_copy(bat_hbm, bat_v)

        def chunk_body(i, carry):
            e0 = wid * EW + i * C
            sl = pl.ds(e0, C)
            lin = [
                pltpu.async_copy(src_hbm.at[sl], src_v, sem_lin),
                pltpu.async_copy(dst_hbm.at[sl], dst_v, sem_lin),
                pltpu.async_copy(s0_hbm.at[sl], h0_v, sem_lin),
                pltpu.async_copy(s1_hbm.at[sl], h1_v, sem_lin),
                pltpu.async_copy(s2_hbm.at[sl], h2_v, sem_lin),
            ]
            lin[0].wait()
            lin[1].wait()
            gth = [
                pltpu.async_copy(px_hbm.at[src_v], sx_v, sem_g),
                pltpu.async_copy(py_hbm.at[src_v], sy_v, sem_g),
                pltpu.async_copy(pz_hbm.at[src_v], sz_v, sem_g),
                pltpu.async_copy(px_hbm.at[dst_v], dx_v, sem_g),
                pltpu.async_copy(py_hbm.at[dst_v], dy_v, sem_g),
                pltpu.async_copy(pz_hbm.at[dst_v], dz_v, sem_g),
            ]
            for g_ in gth:
                g_.wait()
            lin[2].wait()
            lin[3].wait()
            lin[4].wait()

            def grp(g, carry2):
                q = pl.ds(g * _L, _L)
                b9 = plsc.load_gather(bat_v, [src_v[q]]) * 9
                lat = [plsc.load_gather(lat_v, [b9 + kk]) for kk in range(9)]
                sh0 = h0_v[q]
                sh1 = h1_v[q]
                sh2 = h2_v[q]
                vx = dx_v[q] - sx_v[q] + sh0 * lat[0] + sh1 * lat[3] \
                    + sh2 * lat[6]
                vy = dy_v[q] - sy_v[q] + sh0 * lat[1] + sh1 * lat[4] \
                    + sh2 * lat[7]
                vz = dz_v[q] - sz_v[q] + sh0 * lat[2] + sh1 * lat[5] \
                    + sh2 * lat[8]
                n2 = jnp.maximum(vx * vx + vy * vy + vz * vz, 1e-30)
                out_v[q] = n2 * _rsqrt(n2)
                return carry2

            lax.fori_loop(0, R, grp, 0)
            pltpu.async_copy(out_v, out_hbm.at[sl], sem_out).wait()
            return carry

        lax.fori_loop(0, NCH, chunk_body, 0)

    return k


def kernel(pos, edge_index, edge_shift, lattice, batch):
    N = pos.shape[0]
    E = edge_index.shape[1]
    G = lattice.shape[0]
    latf = lattice.reshape(G * 9)
    ew = E // _NW
    c = min(1600, ew)
    c -= c % _L
    while c > _L and ew % c:
        c -= _L
    k = _make_sc_kernel(N, E, G, c)
    return k(pos[:, 0], pos[:, 1], pos[:, 2], batch, latf,
             edge_index[0], edge_index[1],
             edge_shift[:, 0], edge_shift[:, 1], edge_shift[:, 2])
